# baseline (speedup 1.0000x reference)
import jax
import jax.numpy as jnp
from jax import lax
from jax.experimental import pallas as pl
from jax.experimental.pallas import tpu as pltpu
from jax.experimental.pallas import tpu_sc as plsc

B, N, QL, D = 16, 2048, 20, 64
K = 5
L = 16
NCHUNK = N // L
NACC = 4
NPAD = 8

_NEG_INF = float("-inf")


def _sc_body(logits_hbm, idx_hbm, prob_hbm, logit_v, sidx_v, prob8_v):
    c = lax.axis_index("c")
    s = lax.axis_index("s")

    @pl.when(s < 8)
    def _work():
        b = c * 8 + s
        iota = lax.iota(jnp.int32, L)

        pltpu.sync_copy(logits_hbm.at[b], logit_v)

        tv_vec = jnp.zeros((L,), jnp.float32)
        ti_vec = jnp.zeros((L,), jnp.int32)
        for p in range(K):
            def scan_body(i, carry):
                bvs, bis = carry
                nbvs, nbis = [], []
                for a in range(NACC):
                    start = (i * NACC + a) * L
                    v = logit_v[pl.ds(start, L)]
                    gidx = iota + start
                    m = v > bvs[a]
                    nbvs.append(jnp.where(m, v, bvs[a]))
                    nbis.append(jnp.where(m, gidx, bis[a]))
                return (nbvs, nbis)

            init = ([jnp.full((L,), _NEG_INF, jnp.float32)] * NACC,
                    [jnp.zeros((L,), jnp.int32)] * NACC)
            bvs, bis = lax.fori_loop(0, NCHUNK // NACC, scan_body, init)

            def comb(v0, i0, v1, i1):
                m = (v1 > v0) | ((v1 == v0) & (i1 < i0))
                return jnp.where(m, v1, v0), jnp.where(m, i1, i0)

            v01, i01 = comb(bvs[0], bis[0], bvs[1], bis[1])
            v23, i23 = comb(bvs[2], bis[2], bvs[3], bis[3])
            bv, bi = comb(v01, i01, v23, i23)

            mx = jnp.max(bv)
            gi = jnp.min(jnp.where(bv == mx, bi, jnp.int32(N)))

            plsc.store_scatter(logit_v, [jnp.broadcast_to(gi, (L,))],
                               jnp.full((L,), _NEG_INF, jnp.float32),
                               mask=iota == 0)

            tv_vec = jnp.where(iota == p, mx, tv_vec)
            ti_vec = jnp.where(iota == p, gi, ti_vec)

        valid = iota < K
        keys = jnp.where(valid, ti_vec, jnp.int32(N) + iota)
        sk, sv = plsc.sort_key_val(keys, tv_vec)

        mval = jnp.max(jnp.where(valid, sv, _NEG_INF))
        e = jnp.where(valid, jnp.exp(sv - mval), 0.0)
        pr = e / jnp.sum(e)

        sk8 = jnp.where(valid, sk, 0)
        plsc.store_scatter(sidx_v, [iota], sk8, mask=iota < NPAD)
        plsc.store_scatter(prob8_v, [iota], pr, mask=iota < NPAD)
        pltpu.sync_copy(sidx_v, idx_hbm.at[b])
        pltpu.sync_copy(prob8_v, prob_hbm.at[b])


def _build_sc():
    mesh = plsc.VectorSubcoreMesh(core_axis_name="c", subcore_axis_name="s")
    return pl.kernel(
        _sc_body,
        out_type=(jax.ShapeDtypeStruct((B, NPAD), jnp.int32),
                  jax.ShapeDtypeStruct((B, NPAD), jnp.float32)),
        mesh=mesh,
        scratch_types=[
            pltpu.VMEM((N,), jnp.float32),
            pltpu.VMEM((NPAD,), jnp.int32),
            pltpu.VMEM((NPAD,), jnp.float32),
        ],
        compiler_params=pltpu.CompilerParams(needs_layout_passes=False,
                                             use_tc_tiling_on_sc=False),
    )


_sc_topk = _build_sc()


_LW = 128


def _tc_body(idx_ref, prob_ref, *refs):
    x_refs = refs[:K]
    d_refs = refs[K:2 * K]
    o_ref = refs[2 * K]
    b = pl.program_id(0)

    nt = (((1,), (1,)), ((), ()))
    acc = None
    ddsum = None
    for j in range(K):
        n = idx_ref[b, j]
        lane = lax.rem(n, _LW)
        prob = prob_ref[b, j]
        j_row = lax.broadcasted_iota(jnp.int32, (NPAD, _LW), 0) == j
        lane_col = lax.broadcasted_iota(jnp.int32, (NPAD, _LW), 1) == lane
        sel = jnp.where(j_row & lane_col, prob, 0.0)

        x2 = x_refs[j][...].reshape(QL * D, _LW)
        d2 = d_refs[j][...].reshape(QL, _LW)
        y = lax.dot_general(sel, x2, nt, precision=lax.Precision.HIGHEST,
                            preferred_element_type=jnp.float32)
        dd = lax.dot_general(sel, d2, nt, precision=lax.Precision.HIGHEST,
                             preferred_element_type=jnp.float32)
        acc = y if acc is None else acc + y
        ddsum = dd if ddsum is None else ddsum + dd

    e_row = lax.broadcasted_iota(jnp.int32, (QL, QL * D), 0)
    e_col = lax.broadcasted_iota(jnp.int32, (QL, QL * D), 1)
    expand = jnp.where(e_row == e_col // D, 1.0, 0.0)
    ddfull = jnp.dot(ddsum, expand, precision=lax.Precision.HIGHEST,
                     preferred_element_type=jnp.float32)
    o_ref[...] = (acc + ddfull)[:K].reshape(1, K, QL * D)


def _build_tc():
    def _x_map(j):
        return lambda b, idx_ref, prob_ref: (b, 0, 0, idx_ref[b, j] // _LW)

    grid_spec = pltpu.PrefetchScalarGridSpec(
        num_scalar_prefetch=2,
        grid=(B,),
        in_specs=([pl.BlockSpec((1, QL, D, _LW), _x_map(j)) for j in range(K)]
                  + [pl.BlockSpec((1, QL, 1, _LW), _x_map(j))
                     for j in range(K)]),
        out_specs=pl.BlockSpec((1, K, QL * D),
                               lambda b, idx_ref, prob_ref: (b, 0, 0)),
    )
    return pl.pallas_call(
        _tc_body,
        grid_spec=grid_spec,
        out_shape=jax.ShapeDtypeStruct((B, K, QL * D), jnp.float32),
    )


_tc_gather = _build_tc()


def kernel(logits, inputs, dep_hc, k):
    del k
    sidx, probs = _sc_topk(logits)
    x_t = jnp.transpose(inputs, (0, 2, 3, 1))
    d_t = jnp.transpose(dep_hc, (0, 2, 3, 1))
    out = _tc_gather(sidx, probs, *([x_t] * K), *([d_t] * K))
    return out.reshape(B, K, QL, D)


# speedup vs baseline: 1.5732x; 1.5732x over previous
import jax
import jax.numpy as jnp
from jax import lax
from jax.experimental import pallas as pl
from jax.experimental.pallas import tpu as pltpu
from jax.experimental.pallas import tpu_sc as plsc

B, N, QL, D = 16, 2048, 20, 64
K = 5
L = 16
NCHUNK = N // L
NACC = 4
NPAD = 8

_NEG_INF = float("-inf")


def _sc_body(logits_hbm, idx_hbm, prob_hbm, logit_v, sidx_v, prob8_v):
    c = lax.axis_index("c")
    s = lax.axis_index("s")

    @pl.when(s < 8)
    def _work():
        b = c * 8 + s
        iota = lax.iota(jnp.int32, L)

        pltpu.sync_copy(logits_hbm.at[b], logit_v)

        tv_vec = jnp.zeros((L,), jnp.float32)
        ti_vec = jnp.zeros((L,), jnp.int32)
        for p in range(K):
            def scan_body(i, carry):
                bvs, bis = carry
                nbvs, nbis = [], []
                for a in range(NACC):
                    start = (i * NACC + a) * L
                    v = logit_v[pl.ds(start, L)]
                    gidx = iota + start
                    m = v > bvs[a]
                    nbvs.append(jnp.where(m, v, bvs[a]))
                    nbis.append(jnp.where(m, gidx, bis[a]))
                return (nbvs, nbis)

            init = ([jnp.full((L,), _NEG_INF, jnp.float32)] * NACC,
                    [jnp.zeros((L,), jnp.int32)] * NACC)
            bvs, bis = lax.fori_loop(0, NCHUNK // NACC, scan_body, init)

            def comb(v0, i0, v1, i1):
                m = (v1 > v0) | ((v1 == v0) & (i1 < i0))
                return jnp.where(m, v1, v0), jnp.where(m, i1, i0)

            v01, i01 = comb(bvs[0], bis[0], bvs[1], bis[1])
            v23, i23 = comb(bvs[2], bis[2], bvs[3], bis[3])
            bv, bi = comb(v01, i01, v23, i23)

            mx = jnp.max(bv)
            gi = jnp.min(jnp.where(bv == mx, bi, jnp.int32(N)))

            plsc.store_scatter(logit_v, [jnp.broadcast_to(gi, (L,))],
                               jnp.full((L,), _NEG_INF, jnp.float32),
                               mask=iota == 0)

            tv_vec = jnp.where(iota == p, mx, tv_vec)
            ti_vec = jnp.where(iota == p, gi, ti_vec)

        valid = iota < K
        keys = jnp.where(valid, ti_vec, jnp.int32(N) + iota)
        sk, sv = plsc.sort_key_val(keys, tv_vec)

        mval = jnp.max(jnp.where(valid, sv, _NEG_INF))
        e = jnp.where(valid, jnp.exp(sv - mval), 0.0)
        pr = e / jnp.sum(e)

        sk8 = jnp.where(valid, sk, 0)
        plsc.store_scatter(sidx_v, [iota], sk8, mask=iota < NPAD)
        plsc.store_scatter(prob8_v, [iota], pr, mask=iota < NPAD)
        pltpu.sync_copy(sidx_v, idx_hbm.at[b])
        pltpu.sync_copy(prob8_v, prob_hbm.at[b])


def _build_sc():
    mesh = plsc.VectorSubcoreMesh(core_axis_name="c", subcore_axis_name="s")
    return pl.kernel(
        _sc_body,
        out_type=(jax.ShapeDtypeStruct((B, NPAD), jnp.int32),
                  jax.ShapeDtypeStruct((B, NPAD), jnp.float32)),
        mesh=mesh,
        scratch_types=[
            pltpu.VMEM((N,), jnp.float32),
            pltpu.VMEM((NPAD,), jnp.int32),
            pltpu.VMEM((NPAD,), jnp.float32),
        ],
        compiler_params=pltpu.CompilerParams(needs_layout_passes=False,
                                             use_tc_tiling_on_sc=False),
    )


_sc_topk = _build_sc()


_LW = 128


def _tc_body(idx_ref, prob_ref, *refs):
    x_refs = refs[:K]
    d_refs = refs[K:2 * K]
    o_ref = refs[2 * K]
    b = pl.program_id(0)

    nt = (((1,), (1,)), ((), ()))
    acc = None
    ddsum = None
    for j in range(K):
        n = idx_ref[b, j]
        lane = lax.rem(n, _LW)
        prob = prob_ref[b, j]
        j_row = lax.broadcasted_iota(jnp.int32, (NPAD, _LW), 0) == j
        lane_col = lax.broadcasted_iota(jnp.int32, (NPAD, _LW), 1) == lane
        sel = jnp.where(j_row & lane_col, prob, 0.0)

        x2 = x_refs[j][...].reshape(QL * D, _LW)
        d2 = d_refs[j][...].reshape(QL, _LW)
        y = lax.dot_general(sel, x2, nt,
                            preferred_element_type=jnp.float32)
        dd = lax.dot_general(sel, d2, nt,
                             preferred_element_type=jnp.float32)
        acc = y if acc is None else acc + y
        ddsum = dd if ddsum is None else ddsum + dd

    e_row = lax.broadcasted_iota(jnp.int32, (QL, QL * D), 0)
    e_col = lax.broadcasted_iota(jnp.int32, (QL, QL * D), 1)
    expand = jnp.where(e_row == e_col // D, 1.0, 0.0)
    ddfull = jnp.dot(ddsum, expand,
                     preferred_element_type=jnp.float32)
    o_ref[...] = (acc + ddfull)[:K].reshape(1, K, QL * D)


def _build_tc():
    def _x_map(j):
        return lambda b, idx_ref, prob_ref: (b, 0, 0, idx_ref[b, j] // _LW)

    grid_spec = pltpu.PrefetchScalarGridSpec(
        num_scalar_prefetch=2,
        grid=(B,),
        in_specs=([pl.BlockSpec((1, QL, D, _LW), _x_map(j)) for j in range(K)]
                  + [pl.BlockSpec((1, QL, 1, _LW), _x_map(j))
                     for j in range(K)]),
        out_specs=pl.BlockSpec((1, K, QL * D),
                               lambda b, idx_ref, prob_ref: (b, 0, 0)),
    )
    return pl.pallas_call(
        _tc_body,
        grid_spec=grid_spec,
        out_shape=jax.ShapeDtypeStruct((B, K, QL * D), jnp.float32),
    )


_tc_gather = _build_tc()


def kernel(logits, inputs, dep_hc, k):
    del k
    sidx, probs = _sc_topk(logits)
    x_t = jnp.transpose(inputs, (0, 2, 3, 1))
    d_t = jnp.transpose(dep_hc, (0, 2, 3, 1))
    out = _tc_gather(sidx, probs, *([x_t] * K), *([d_t] * K))
    return out.reshape(B, K, QL, D)


# speedup vs baseline: 1.6652x; 1.0585x over previous
import jax
import jax.numpy as jnp
from jax import lax
from jax.experimental import pallas as pl
from jax.experimental.pallas import tpu as pltpu
from jax.experimental.pallas import tpu_sc as plsc

B, N, QL, D = 16, 2048, 20, 64
K = 5
L = 16
NCHUNK = N // L
NACC = 4
NPAD = 8

_NEG_INF = float("-inf")


def _sc_body(logits_hbm, idx_hbm, prob_hbm, logit_v, sidx_v, prob8_v):
    c = lax.axis_index("c")
    s = lax.axis_index("s")

    @pl.when(s < 8)
    def _work():
        b = c * 8 + s
        iota = lax.iota(jnp.int32, L)

        pltpu.sync_copy(logits_hbm.at[b], logit_v)

        tv_vec = jnp.zeros((L,), jnp.float32)
        ti_vec = jnp.zeros((L,), jnp.int32)
        for p in range(K):
            def scan_body(i, carry):
                bvs, bis = carry
                nbvs, nbis = [], []
                for a in range(NACC):
                    start = (i * NACC + a) * L
                    v = logit_v[pl.ds(start, L)]
                    gidx = iota + start
                    m = v > bvs[a]
                    nbvs.append(jnp.where(m, v, bvs[a]))
                    nbis.append(jnp.where(m, gidx, bis[a]))
                return (nbvs, nbis)

            init = ([jnp.full((L,), _NEG_INF, jnp.float32)] * NACC,
                    [jnp.zeros((L,), jnp.int32)] * NACC)
            bvs, bis = lax.fori_loop(0, NCHUNK // NACC, scan_body, init)

            def comb(v0, i0, v1, i1):
                m = (v1 > v0) | ((v1 == v0) & (i1 < i0))
                return jnp.where(m, v1, v0), jnp.where(m, i1, i0)

            v01, i01 = comb(bvs[0], bis[0], bvs[1], bis[1])
            v23, i23 = comb(bvs[2], bis[2], bvs[3], bis[3])
            bv, bi = comb(v01, i01, v23, i23)

            mx = jnp.max(bv)
            gi = jnp.min(jnp.where(bv == mx, bi, jnp.int32(N)))

            plsc.store_scatter(logit_v, [jnp.broadcast_to(gi, (L,))],
                               jnp.full((L,), _NEG_INF, jnp.float32),
                               mask=iota == 0)

            tv_vec = jnp.where(iota == p, mx, tv_vec)
            ti_vec = jnp.where(iota == p, gi, ti_vec)

        valid = iota < K
        keys = jnp.where(valid, ti_vec, jnp.int32(N) + iota)
        sk, sv = plsc.sort_key_val(keys, tv_vec)

        mval = jnp.max(jnp.where(valid, sv, _NEG_INF))
        e = jnp.where(valid, jnp.exp(sv - mval), 0.0)
        pr = e / jnp.sum(e)

        sk8 = jnp.where(valid, sk, 0)
        plsc.store_scatter(sidx_v, [iota], sk8, mask=iota < NPAD)
        plsc.store_scatter(prob8_v, [iota], pr, mask=iota < NPAD)
        pltpu.sync_copy(sidx_v, idx_hbm.at[pl.ds(b * NPAD, NPAD)])
        pltpu.sync_copy(prob8_v, prob_hbm.at[pl.ds(b * NPAD, NPAD)])


def _build_sc():
    mesh = plsc.VectorSubcoreMesh(core_axis_name="c", subcore_axis_name="s")
    return pl.kernel(
        _sc_body,
        out_type=(jax.ShapeDtypeStruct((B * NPAD,), jnp.int32),
                  jax.ShapeDtypeStruct((B * NPAD,), jnp.float32)),
        mesh=mesh,
        scratch_types=[
            pltpu.VMEM((N,), jnp.float32),
            pltpu.VMEM((NPAD,), jnp.int32),
            pltpu.VMEM((NPAD,), jnp.float32),
        ],
        compiler_params=pltpu.CompilerParams(needs_layout_passes=False,
                                             use_tc_tiling_on_sc=False),
    )


_sc_topk = _build_sc()


_LW = 128


def _tc_body(idx_ref, prob_ref, *refs):
    x_refs = refs[:K]
    d_refs = refs[K:2 * K]
    o_ref = refs[2 * K]
    b = pl.program_id(0)

    nt = (((1,), (1,)), ((), ()))
    acc = None
    ddsum = None
    for j in range(K):
        n = idx_ref[b * NPAD + j]
        lane = lax.rem(n, _LW)
        prob = prob_ref[b * NPAD + j]
        j_row = lax.broadcasted_iota(jnp.int32, (NPAD, _LW), 0) == j
        lane_col = lax.broadcasted_iota(jnp.int32, (NPAD, _LW), 1) == lane
        sel = jnp.where(j_row & lane_col, prob, 0.0)

        x2 = x_refs[j][...].reshape(QL * D, _LW)
        d2 = d_refs[j][...].reshape(QL, _LW)
        y = lax.dot_general(sel, x2, nt,
                            preferred_element_type=jnp.float32)
        dd = lax.dot_general(sel, d2, nt,
                             preferred_element_type=jnp.float32)
        acc = y if acc is None else acc + y
        ddsum = dd if ddsum is None else ddsum + dd

    e_row = lax.broadcasted_iota(jnp.int32, (QL, QL * D), 0)
    e_col = lax.broadcasted_iota(jnp.int32, (QL, QL * D), 1)
    expand = jnp.where(e_row == e_col // D, 1.0, 0.0)
    ddfull = jnp.dot(ddsum, expand,
                     preferred_element_type=jnp.float32)
    o_ref[...] = (acc + ddfull)[:K].reshape(1, K, QL * D)


def _build_tc():
    def _x_map(j):
        return lambda b, idx_ref, prob_ref: (b, 0, 0,
                                             idx_ref[b * NPAD + j] // _LW)

    grid_spec = pltpu.PrefetchScalarGridSpec(
        num_scalar_prefetch=2,
        grid=(B,),
        in_specs=([pl.BlockSpec((1, QL, D, _LW), _x_map(j)) for j in range(K)]
                  + [pl.BlockSpec((1, QL, 1, _LW), _x_map(j))
                     for j in range(K)]),
        out_specs=pl.BlockSpec((1, K, QL * D),
                               lambda b, idx_ref, prob_ref: (b, 0, 0)),
    )
    return pl.pallas_call(
        _tc_body,
        grid_spec=grid_spec,
        out_shape=jax.ShapeDtypeStruct((B, K, QL * D), jnp.float32),
    )


_tc_gather = _build_tc()


def kernel(logits, inputs, dep_hc, k):
    del k
    sidx, probs = _sc_topk(logits)
    x_t = jnp.transpose(inputs, (0, 2, 3, 1))
    d_t = jnp.transpose(dep_hc, (0, 2, 3, 1))
    out = _tc_gather(sidx, probs, *([x_t] * K), *([d_t] * K))
    return out.reshape(B, K, QL, D)


# speedup vs baseline: 1.6766x; 1.0069x over previous
import jax
import jax.numpy as jnp
from jax import lax
from jax.experimental import pallas as pl
from jax.experimental.pallas import tpu as pltpu
from jax.experimental.pallas import tpu_sc as plsc

B, N, QL, D = 16, 2048, 20, 64
K = 5
L = 16
NCHUNK = N // L
NACC = 4
NPAD = 8

_NEG_INF = float("-inf")


def _sc_body(logits_hbm, idx_hbm, prob_hbm, logit_v, sidx_v, prob8_v):
    c = lax.axis_index("c")
    s = lax.axis_index("s")

    @pl.when(s < 8)
    def _work():
        b = c * 8 + s
        iota = lax.iota(jnp.int32, L)

        pltpu.sync_copy(logits_hbm.at[b], logit_v)

        def pass_body(p, tvti):
            tv_vec, ti_vec = tvti

            def scan_body(i, carry):
                bvs, bis = carry
                nbvs, nbis = [], []
                for a in range(NACC):
                    start = (i * NACC + a) * L
                    v = logit_v[pl.ds(start, L)]
                    gidx = iota + start
                    m = v > bvs[a]
                    nbvs.append(jnp.where(m, v, bvs[a]))
                    nbis.append(jnp.where(m, gidx, bis[a]))
                return (nbvs, nbis)

            init = ([jnp.full((L,), _NEG_INF, jnp.float32)] * NACC,
                    [jnp.zeros((L,), jnp.int32)] * NACC)
            bvs, bis = lax.fori_loop(0, NCHUNK // NACC, scan_body, init)

            def comb(v0, i0, v1, i1):
                m = (v1 > v0) | ((v1 == v0) & (i1 < i0))
                return jnp.where(m, v1, v0), jnp.where(m, i1, i0)

            v01, i01 = comb(bvs[0], bis[0], bvs[1], bis[1])
            v23, i23 = comb(bvs[2], bis[2], bvs[3], bis[3])
            bv, bi = comb(v01, i01, v23, i23)

            mx = jnp.max(bv)
            gi = jnp.min(jnp.where(bv == mx, bi, jnp.int32(N)))

            plsc.store_scatter(logit_v, [jnp.broadcast_to(gi, (L,))],
                               jnp.full((L,), _NEG_INF, jnp.float32),
                               mask=iota == 0)

            return (jnp.where(iota == p, mx, tv_vec),
                    jnp.where(iota == p, gi, ti_vec))

        tv_vec, ti_vec = lax.fori_loop(
            0, K, pass_body,
            (jnp.zeros((L,), jnp.float32), jnp.zeros((L,), jnp.int32)))

        valid = iota < K
        keys = jnp.where(valid, ti_vec, jnp.int32(N) + iota)
        sk, sv = plsc.sort_key_val(keys, tv_vec)

        mval = jnp.max(jnp.where(valid, sv, _NEG_INF))
        e = jnp.where(valid, jnp.exp(sv - mval), 0.0)
        pr = e / jnp.sum(e)

        sk8 = jnp.where(valid, sk, 0)
        plsc.store_scatter(sidx_v, [iota], sk8, mask=iota < NPAD)
        plsc.store_scatter(prob8_v, [iota], pr, mask=iota < NPAD)
        pltpu.sync_copy(sidx_v, idx_hbm.at[pl.ds(b * NPAD, NPAD)])
        pltpu.sync_copy(prob8_v, prob_hbm.at[pl.ds(b * NPAD, NPAD)])


def _build_sc():
    mesh = plsc.VectorSubcoreMesh(core_axis_name="c", subcore_axis_name="s")
    return pl.kernel(
        _sc_body,
        out_type=(jax.ShapeDtypeStruct((B * NPAD,), jnp.int32),
                  jax.ShapeDtypeStruct((B * NPAD,), jnp.float32)),
        mesh=mesh,
        scratch_types=[
            pltpu.VMEM((N,), jnp.float32),
            pltpu.VMEM((NPAD,), jnp.int32),
            pltpu.VMEM((NPAD,), jnp.float32),
        ],
        compiler_params=pltpu.CompilerParams(needs_layout_passes=False,
                                             use_tc_tiling_on_sc=False),
    )


_sc_topk = _build_sc()


_LW = 128


def _tc_body(idx_ref, prob_ref, *refs):
    x_refs = refs[:K]
    d_refs = refs[K:2 * K]
    o_ref = refs[2 * K]
    b = pl.program_id(0)

    nt = (((1,), (1,)), ((), ()))
    acc = None
    ddsum = None
    for j in range(K):
        n = idx_ref[b * NPAD + j]
        lane = lax.rem(n, _LW)
        prob = prob_ref[b * NPAD + j]
        j_row = lax.broadcasted_iota(jnp.int32, (NPAD, _LW), 0) == j
        lane_col = lax.broadcasted_iota(jnp.int32, (NPAD, _LW), 1) == lane
        sel = jnp.where(j_row & lane_col, prob, 0.0)

        x2 = x_refs[j][...].reshape(QL * D, _LW)
        d2 = d_refs[j][...].reshape(QL, _LW)
        y = lax.dot_general(sel, x2, nt,
                            preferred_element_type=jnp.float32)
        dd = lax.dot_general(sel, d2, nt,
                             preferred_element_type=jnp.float32)
        acc = y if acc is None else acc + y
        ddsum = dd if ddsum is None else ddsum + dd

    e_row = lax.broadcasted_iota(jnp.int32, (QL, QL * D), 0)
    e_col = lax.broadcasted_iota(jnp.int32, (QL, QL * D), 1)
    expand = jnp.where(e_row == e_col // D, 1.0, 0.0)
    ddfull = jnp.dot(ddsum, expand,
                     preferred_element_type=jnp.float32)
    o_ref[...] = (acc + ddfull)[:K].reshape(1, K, QL * D)


def _build_tc():
    def _x_map(j):
        return lambda b, idx_ref, prob_ref: (b, 0, 0,
                                             idx_ref[b * NPAD + j] // _LW)

    grid_spec = pltpu.PrefetchScalarGridSpec(
        num_scalar_prefetch=2,
        grid=(B,),
        in_specs=([pl.BlockSpec((1, QL, D, _LW), _x_map(j)) for j in range(K)]
                  + [pl.BlockSpec((1, QL, 1, _LW), _x_map(j))
                     for j in range(K)]),
        out_specs=pl.BlockSpec((1, K, QL * D),
                               lambda b, idx_ref, prob_ref: (b, 0, 0)),
    )
    return pl.pallas_call(
        _tc_body,
        grid_spec=grid_spec,
        out_shape=jax.ShapeDtypeStruct((B, K, QL * D), jnp.float32),
    )


_tc_gather = _build_tc()


def kernel(logits, inputs, dep_hc, k):
    del k
    sidx, probs = _sc_topk(logits)
    x_t = jnp.transpose(inputs, (0, 2, 3, 1))
    d_t = jnp.transpose(dep_hc, (0, 2, 3, 1))
    out = _tc_gather(sidx, probs, *([x_t] * K), *([d_t] * K))
    return out.reshape(B, K, QL, D)
